# Initial kernel scaffold; baseline (speedup 1.0000x reference)
import functools

import jax
import jax.numpy as jnp
from jax import lax
from jax.experimental import pallas as pl
from jax.experimental.pallas import tpu as pltpu
from jax.experimental.pallas import tpu_sc as plsc

N = 10000
E = 320000
F = 128
G = 64
NC = 2
NS = 16
NW = NC * NS
EC = E // NW
KB = 80
NBLK = EC // KB
RPT = N // NS
ZR = 125

_MESH = plsc.VectorSubcoreMesh(core_axis_name="c", subcore_axis_name="s")


@functools.partial(
    pl.kernel,
    out_type=jax.ShapeDtypeStruct((NW, N), jnp.float32),
    mesh=_MESH,
    scratch_types=[
        pltpu.VMEM((EC,), jnp.int32),
        pltpu.VMEM((EC,), jnp.float32),
        pltpu.VMEM((N,), jnp.float32),
    ],
)
def _deg_sc(dst_hbm, w_hbm, out_hbm, didx, wbuf, acc):
    c = lax.axis_index("c")
    s = lax.axis_index("s")
    wid = s * NC + c

    def zero(i, _):
        acc[pl.ds(i * 16, 16)] = jnp.zeros((16,), jnp.float32)
        return 0

    lax.fori_loop(0, N // 16, zero, 0)
    pltpu.sync_copy(dst_hbm.at[wid], didx)
    pltpu.sync_copy(w_hbm.at[wid], wbuf)

    def edge(e, _):
        d = didx[e]
        acc[d] = acc[d] + wbuf[e]
        return 0

    lax.fori_loop(0, EC, edge, 0)
    pltpu.sync_copy(acc, out_hbm.at[wid])


@functools.partial(
    pl.kernel,
    out_type=jax.ShapeDtypeStruct((NC, N, F), jnp.float32),
    mesh=_MESH,
    scratch_types=[
        pltpu.VMEM((NBLK, KB), jnp.int32),
        pltpu.VMEM((NBLK, KB), jnp.int32),
        pltpu.VMEM((NBLK, KB), jnp.float32),
        pltpu.VMEM((KB, F), jnp.float32),
        pltpu.VMEM((ZR, F), jnp.float32),
        pltpu.VMEM_SHARED((N, F), jnp.float32),
        pltpu.SemaphoreType.DMA,
    ],
)
def _spmm_sc(u_hbm, src_hbm, dst_hbm, w_hbm, out_hbm,
             sidx, didx, wblk, rows, zrows, accum, sem):
    c = lax.axis_index("c")
    s = lax.axis_index("s")
    wid = s * NC + c

    def zrow(i, _):
        for j in range(F // 16):
            zrows[i, pl.ds(j * 16, 16)] = jnp.zeros((16,), jnp.float32)
        return 0

    lax.fori_loop(0, ZR, zrow, 0)
    for j in range(RPT // ZR):
        pltpu.sync_copy(zrows, accum.at[pl.ds(s * RPT + j * ZR, ZR)])
    plsc.subcore_barrier()

    pltpu.sync_copy(src_hbm.at[wid], sidx)
    pltpu.sync_copy(dst_hbm.at[wid], didx)
    pltpu.sync_copy(w_hbm.at[wid], wblk)

    def block(b, _):
        pltpu.async_copy(u_hbm.at[sidx.at[b]], rows, sem).wait()

        def scale(r, _):
            wv = wblk[b, r]
            for j in range(F // 16):
                rows[r, pl.ds(j * 16, 16)] = rows[r, pl.ds(j * 16, 16)] * wv
            return 0

        lax.fori_loop(0, KB, scale, 0)
        pltpu.sync_copy(rows, accum.at[didx.at[b]], add=True)
        return 0

    lax.fori_loop(0, NBLK, block, 0)
    plsc.subcore_barrier()
    pltpu.sync_copy(accum.at[pl.ds(s * RPT, RPT)],
                    out_hbm.at[c, pl.ds(s * RPT, RPT)])


_BN = 1000


def _k2_body(degsT_ref, x_ref, w_ref, u_ref, dis_ref):
    deg = jnp.sum(degsT_ref[...], axis=1, keepdims=True) + 1.0
    dis = lax.rsqrt(deg)
    dis_ref[...] = dis
    u_ref[...] = dis * jnp.dot(x_ref[...], w_ref[...],
                               preferred_element_type=jnp.float32)


_k2 = pl.pallas_call(
    _k2_body,
    grid=(N // _BN,),
    in_specs=[
        pl.BlockSpec((_BN, NW), lambda i: (i, 0)),
        pl.BlockSpec((_BN, F), lambda i: (i, 0)),
        pl.BlockSpec((F, F), lambda i: (0, 0)),
    ],
    out_specs=[
        pl.BlockSpec((_BN, F), lambda i: (i, 0)),
        pl.BlockSpec((_BN, 1), lambda i: (i, 0)),
    ],
    out_shape=[
        jax.ShapeDtypeStruct((N, F), jnp.float32),
        jax.ShapeDtypeStruct((N, 1), jnp.float32),
    ],
)


def _leaky(x):
    return jnp.where(x >= 0, x, 0.01 * x)


def _k4_body(dis_ref, p_ref, u_ref, b_ref, w2_ref, u2_ref):
    dis = dis_ref[...]
    pre = dis * (p_ref[0] + p_ref[1] + u_ref[...]) + b_ref[...]
    h = _leaky(pre)
    u2_ref[...] = dis * jnp.dot(h, w2_ref[...],
                                preferred_element_type=jnp.float32)


_k4 = pl.pallas_call(
    _k4_body,
    grid=(N // _BN,),
    in_specs=[
        pl.BlockSpec((_BN, 1), lambda i: (i, 0)),
        pl.BlockSpec((NC, _BN, F), lambda i: (0, i, 0)),
        pl.BlockSpec((_BN, F), lambda i: (i, 0)),
        pl.BlockSpec((1, F), lambda i: (0, 0)),
        pl.BlockSpec((F, F), lambda i: (0, 0)),
    ],
    out_specs=pl.BlockSpec((_BN, F), lambda i: (i, 0)),
    out_shape=jax.ShapeDtypeStruct((N, F), jnp.float32),
)


def _k6_body(dis_ref, p_ref, u_ref, b_ref, bat_ref, sums_ref, cnts_ref):
    i = pl.program_id(0)
    dis = dis_ref[...]
    pre = dis * (p_ref[0] + p_ref[1] + u_ref[...]) + b_ref[...]
    h = _leaky(pre)
    gids = lax.broadcasted_iota(jnp.int32, (_BN, G), 1)
    oh = (bat_ref[...] == gids).astype(jnp.float32)
    psum = lax.dot_general(oh, h, (((0,), (0,)), ((), ())),
                           preferred_element_type=jnp.float32)
    pcnt = lax.dot_general(oh, jnp.ones((_BN, 1), jnp.float32),
                           (((0,), (0,)), ((), ())),
                           preferred_element_type=jnp.float32)

    @pl.when(i == 0)
    def _():
        sums_ref[...] = jnp.zeros_like(sums_ref)
        cnts_ref[...] = jnp.zeros_like(cnts_ref)

    sums_ref[...] += psum
    cnts_ref[...] += pcnt


_k6 = pl.pallas_call(
    _k6_body,
    grid=(N // _BN,),
    in_specs=[
        pl.BlockSpec((_BN, 1), lambda i: (i, 0)),
        pl.BlockSpec((NC, _BN, F), lambda i: (0, i, 0)),
        pl.BlockSpec((_BN, F), lambda i: (i, 0)),
        pl.BlockSpec((1, F), lambda i: (0, 0)),
        pl.BlockSpec((_BN, 1), lambda i: (i, 0)),
    ],
    out_specs=[
        pl.BlockSpec((G, F), lambda i: (0, 0)),
        pl.BlockSpec((G, 1), lambda i: (0, 0)),
    ],
    out_shape=[
        jax.ShapeDtypeStruct((G, F), jnp.float32),
        jax.ShapeDtypeStruct((G, 1), jnp.float32),
    ],
)


def _k7_body(sums_ref, cnts_ref,
             cw1, cb1, cw2, cb2, cw3, cb3,
             rw1, rb1, rw2, rb2, rw3, rb3,
             chi_ref, rp_ref):
    pooled = sums_ref[...] / jnp.maximum(cnts_ref[...], 1.0)

    def head(W1r, B1r, W2r, B2r, W3r, B3r):
        a = jnp.dot(pooled, W1r[...], preferred_element_type=jnp.float32)
        a = _leaky(a + B1r[...])
        a = jnp.dot(a, W2r[...], preferred_element_type=jnp.float32)
        a = _leaky(a + B2r[...])
        return jnp.dot(a, W3r[...], preferred_element_type=jnp.float32) + B3r[...]

    chi_ref[...] = head(cw1, cb1, cw2, cb2, cw3, cb3)
    rp_ref[...] = head(rw1, rb1, rw2, rb2, rw3, rb3)


_k7 = pl.pallas_call(
    _k7_body,
    out_shape=[
        jax.ShapeDtypeStruct((G, 1), jnp.float32),
        jax.ShapeDtypeStruct((G, 1), jnp.float32),
    ],
)


def kernel(X, Edge_index, Edge_weight, Batching,
           W1, b1, W2, b2,
           cW1, cb1, cW2, cb2, cW3, cb3,
           rW1, rb1, rW2, rb2, rW3, rb3):
    src = Edge_index[0].reshape(NW, NBLK, KB)
    dst = Edge_index[1].reshape(NW, NBLK, KB)
    w3 = Edge_weight.reshape(NW, NBLK, KB)
    dstf = Edge_index[1].reshape(NW, EC)
    wf = Edge_weight.reshape(NW, EC)

    degs = _deg_sc(dstf, wf)
    degsT = degs.T
    u1, dis = _k2(degsT, X, W1)

    p1 = _spmm_sc(u1, src, dst, w3)
    u2 = _k4(dis, p1, u1, b1.reshape(1, F), W2)

    p2 = _spmm_sc(u2, src, dst, w3)
    sums, cnts = _k6(dis, p2, u2, b2.reshape(1, F),
                     Batching.reshape(N, 1).astype(jnp.int32))

    chi, rp = _k7(sums, cnts,
                  cW1, cb1.reshape(1, -1), cW2, cb2.reshape(1, -1),
                  cW3, cb3.reshape(1, -1),
                  rW1, rb1.reshape(1, -1), rW2, rb2.reshape(1, -1),
                  rW3, rb3.reshape(1, -1))
    return jnp.concatenate((chi, rp), axis=1)


# speedup vs baseline: 7.1943x; 7.1943x over previous
import functools

import jax
import jax.numpy as jnp
from jax import lax
from jax.experimental import pallas as pl
from jax.experimental.pallas import tpu as pltpu
from jax.experimental.pallas import tpu_sc as plsc

N = 10000
E = 320000
F = 128
G = 64
NC = 2
NS = 16
NW = NC * NS
EC = E // NW
KB = 80
NBLK = EC // KB
RPT = N // NS
ZR = 125

_MESH = plsc.VectorSubcoreMesh(core_axis_name="c", subcore_axis_name="s")


@functools.partial(
    pl.kernel,
    out_type=jax.ShapeDtypeStruct((NW, N), jnp.float32),
    mesh=_MESH,
    compiler_params=pltpu.CompilerParams(use_tc_tiling_on_sc=False),
    scratch_types=[
        pltpu.VMEM((EC,), jnp.int32),
        pltpu.VMEM((EC,), jnp.float32),
        pltpu.VMEM((N + 16,), jnp.float32),
    ],
)
def _deg_sc(dst_hbm, w_hbm, out_hbm, didx, wbuf, acc):
    c = lax.axis_index("c")
    s = lax.axis_index("s")
    wid = s * NC + c

    def zero(i, _):
        acc[pl.ds(i * 16, 16)] = jnp.zeros((16,), jnp.float32)
        return 0

    lax.fori_loop(0, (N + 16) // 16, zero, 0)
    pltpu.sync_copy(dst_hbm.at[wid], didx)
    pltpu.sync_copy(w_hbm.at[wid], wbuf)

    lane0 = lax.iota(jnp.int32, 16) == 0

    def edge16(e, _):
        dvec = didx[pl.ds(e * 16, 16)]
        wvec = wbuf[pl.ds(e * 16, 16)]
        for lane in range(16):
            d = dvec[lane]
            inc = jnp.where(lane0, wvec[lane], 0.0)
            acc[pl.ds(d, 16)] = acc[pl.ds(d, 16)] + inc
        return 0

    lax.fori_loop(0, EC // 16, edge16, 0)
    pltpu.sync_copy(acc.at[pl.ds(0, N)], out_hbm.at[wid])


FH = F // 2


@functools.partial(
    pl.kernel,
    out_type=jax.ShapeDtypeStruct((NC, N, FH), jnp.float32),
    mesh=_MESH,
    compiler_params=pltpu.CompilerParams(use_tc_tiling_on_sc=False),
    scratch_types=[
        pltpu.VMEM((NBLK, KB), jnp.int32),
        pltpu.VMEM((NBLK, KB), jnp.int32),
        pltpu.VMEM((NBLK, KB), jnp.float32),
        pltpu.VMEM((KB, FH), jnp.float32),
        pltpu.VMEM((ZR, FH), jnp.float32),
        pltpu.VMEM_SHARED((N, FH), jnp.float32),
        pltpu.SemaphoreType.DMA,
    ],
)
def _spmm_sc(u_hbm, src_hbm, dst_hbm, w_hbm, out_hbm,
             sidx, didx, wblk, rows, zrows, accum, sem):
    c = lax.axis_index("c")
    s = lax.axis_index("s")
    wid = s * NC + c

    def zrow(i, _):
        for j in range(FH // 16):
            zrows[i, pl.ds(j * 16, 16)] = jnp.zeros((16,), jnp.float32)
        return 0

    lax.fori_loop(0, ZR, zrow, 0)
    for j in range(RPT // ZR):
        pltpu.sync_copy(zrows, accum.at[pl.ds(s * RPT + j * ZR, ZR)])
    plsc.subcore_barrier()

    pltpu.sync_copy(src_hbm.at[wid], sidx)
    pltpu.sync_copy(dst_hbm.at[wid], didx)
    pltpu.sync_copy(w_hbm.at[wid], wblk)

    def block(b, _):
        pltpu.async_copy(u_hbm.at[sidx.at[b]], rows, sem).wait()

        def scale16(r16, _):
            wvec = wblk[b, pl.ds(r16 * 16, 16)]
            for lane in range(16):
                wv = wvec[lane]
                r = r16 * 16 + lane
                for j in range(FH // 16):
                    rows[r, pl.ds(j * 16, 16)] = rows[r, pl.ds(j * 16, 16)] * wv
            return 0

        lax.fori_loop(0, KB // 16, scale16, 0)
        pltpu.sync_copy(rows, accum.at[didx.at[b]], add=True)
        return 0

    lax.fori_loop(0, NBLK, block, 0)
    plsc.subcore_barrier()
    pltpu.sync_copy(accum.at[pl.ds(s * RPT, RPT)],
                    out_hbm.at[c, pl.ds(s * RPT, RPT)])


_BN = 1000


def _k2_body(degsT_ref, x_ref, wlo_ref, whi_ref, ulo_ref, uhi_ref, dis_ref):
    deg = jnp.sum(degsT_ref[...], axis=1, keepdims=True) + 1.0
    dis = lax.rsqrt(deg)
    dis_ref[...] = dis
    x = x_ref[...]
    ulo_ref[...] = dis * jnp.dot(x, wlo_ref[...],
                                 preferred_element_type=jnp.float32)
    uhi_ref[...] = dis * jnp.dot(x, whi_ref[...],
                                 preferred_element_type=jnp.float32)


_k2 = pl.pallas_call(
    _k2_body,
    grid=(N // _BN,),
    in_specs=[
        pl.BlockSpec((_BN, NW), lambda i: (i, 0)),
        pl.BlockSpec((_BN, F), lambda i: (i, 0)),
        pl.BlockSpec((F, FH), lambda i: (0, 0)),
        pl.BlockSpec((F, FH), lambda i: (0, 0)),
    ],
    out_specs=[
        pl.BlockSpec((_BN, FH), lambda i: (i, 0)),
        pl.BlockSpec((_BN, FH), lambda i: (i, 0)),
        pl.BlockSpec((_BN, 1), lambda i: (i, 0)),
    ],
    out_shape=[
        jax.ShapeDtypeStruct((N, FH), jnp.float32),
        jax.ShapeDtypeStruct((N, FH), jnp.float32),
        jax.ShapeDtypeStruct((N, 1), jnp.float32),
    ],
)


def _leaky(x):
    return jnp.where(x >= 0, x, 0.01 * x)


def _h_from_halves(dis, plo_ref, phi_ref, ulo_ref, uhi_ref, b_ref):
    hlo = dis * (plo_ref[0] + plo_ref[1] + ulo_ref[...])
    hhi = dis * (phi_ref[0] + phi_ref[1] + uhi_ref[...])
    pre = jnp.concatenate((hlo, hhi), axis=1) + b_ref[...]
    return _leaky(pre)


def _k4_body(dis_ref, plo_ref, phi_ref, ulo_ref, uhi_ref, b_ref,
             w2lo_ref, w2hi_ref, u2lo_ref, u2hi_ref):
    dis = dis_ref[...]
    h = _h_from_halves(dis, plo_ref, phi_ref, ulo_ref, uhi_ref, b_ref)
    u2lo_ref[...] = dis * jnp.dot(h, w2lo_ref[...],
                                  preferred_element_type=jnp.float32)
    u2hi_ref[...] = dis * jnp.dot(h, w2hi_ref[...],
                                  preferred_element_type=jnp.float32)


_k4 = pl.pallas_call(
    _k4_body,
    grid=(N // _BN,),
    in_specs=[
        pl.BlockSpec((_BN, 1), lambda i: (i, 0)),
        pl.BlockSpec((NC, _BN, FH), lambda i: (0, i, 0)),
        pl.BlockSpec((NC, _BN, FH), lambda i: (0, i, 0)),
        pl.BlockSpec((_BN, FH), lambda i: (i, 0)),
        pl.BlockSpec((_BN, FH), lambda i: (i, 0)),
        pl.BlockSpec((1, F), lambda i: (0, 0)),
        pl.BlockSpec((F, FH), lambda i: (0, 0)),
        pl.BlockSpec((F, FH), lambda i: (0, 0)),
    ],
    out_specs=[
        pl.BlockSpec((_BN, FH), lambda i: (i, 0)),
        pl.BlockSpec((_BN, FH), lambda i: (i, 0)),
    ],
    out_shape=[
        jax.ShapeDtypeStruct((N, FH), jnp.float32),
        jax.ShapeDtypeStruct((N, FH), jnp.float32),
    ],
)


def _k6_body(dis_ref, plo_ref, phi_ref, ulo_ref, uhi_ref, b_ref, bat_ref,
             sums_ref, cnts_ref):
    i = pl.program_id(0)
    dis = dis_ref[...]
    h = _h_from_halves(dis, plo_ref, phi_ref, ulo_ref, uhi_ref, b_ref)
    gids = lax.broadcasted_iota(jnp.int32, (_BN, G), 1)
    oh = (bat_ref[...] == gids).astype(jnp.float32)
    psum = lax.dot_general(oh, h, (((0,), (0,)), ((), ())),
                           preferred_element_type=jnp.float32)
    pcnt = lax.dot_general(oh, jnp.ones((_BN, 1), jnp.float32),
                           (((0,), (0,)), ((), ())),
                           preferred_element_type=jnp.float32)

    @pl.when(i == 0)
    def _():
        sums_ref[...] = jnp.zeros_like(sums_ref)
        cnts_ref[...] = jnp.zeros_like(cnts_ref)

    sums_ref[...] += psum
    cnts_ref[...] += pcnt


_k6 = pl.pallas_call(
    _k6_body,
    grid=(N // _BN,),
    in_specs=[
        pl.BlockSpec((_BN, 1), lambda i: (i, 0)),
        pl.BlockSpec((NC, _BN, FH), lambda i: (0, i, 0)),
        pl.BlockSpec((NC, _BN, FH), lambda i: (0, i, 0)),
        pl.BlockSpec((_BN, FH), lambda i: (i, 0)),
        pl.BlockSpec((_BN, FH), lambda i: (i, 0)),
        pl.BlockSpec((1, F), lambda i: (0, 0)),
        pl.BlockSpec((_BN, 1), lambda i: (i, 0)),
    ],
    out_specs=[
        pl.BlockSpec((G, F), lambda i: (0, 0)),
        pl.BlockSpec((G, 1), lambda i: (0, 0)),
    ],
    out_shape=[
        jax.ShapeDtypeStruct((G, F), jnp.float32),
        jax.ShapeDtypeStruct((G, 1), jnp.float32),
    ],
)


def _k7_body(sums_ref, cnts_ref,
             cw1, cb1, cw2, cb2, cw3, cb3,
             rw1, rb1, rw2, rb2, rw3, rb3,
             chi_ref, rp_ref):
    pooled = sums_ref[...] / jnp.maximum(cnts_ref[...], 1.0)

    def head(W1r, B1r, W2r, B2r, W3r, B3r):
        a = jnp.dot(pooled, W1r[...], preferred_element_type=jnp.float32)
        a = _leaky(a + B1r[...])
        a = jnp.dot(a, W2r[...], preferred_element_type=jnp.float32)
        a = _leaky(a + B2r[...])
        return jnp.dot(a, W3r[...], preferred_element_type=jnp.float32) + B3r[...]

    chi_ref[...] = head(cw1, cb1, cw2, cb2, cw3, cb3)
    rp_ref[...] = head(rw1, rb1, rw2, rb2, rw3, rb3)


_k7 = pl.pallas_call(
    _k7_body,
    out_shape=[
        jax.ShapeDtypeStruct((G, 1), jnp.float32),
        jax.ShapeDtypeStruct((G, 1), jnp.float32),
    ],
)


def kernel(X, Edge_index, Edge_weight, Batching,
           W1, b1, W2, b2,
           cW1, cb1, cW2, cb2, cW3, cb3,
           rW1, rb1, rW2, rb2, rW3, rb3):
    src = Edge_index[0].reshape(NW, NBLK, KB)
    dst = Edge_index[1].reshape(NW, NBLK, KB)
    w3 = Edge_weight.reshape(NW, NBLK, KB)
    dstf = Edge_index[1].reshape(NW, EC)
    wf = Edge_weight.reshape(NW, EC)

    degs = _deg_sc(dstf, wf)
    degsT = degs.T
    u1lo, u1hi, dis = _k2(degsT, X, W1[:, :FH], W1[:, FH:])

    p1lo = _spmm_sc(u1lo, src, dst, w3)
    p1hi = _spmm_sc(u1hi, src, dst, w3)
    u2lo, u2hi = _k4(dis, p1lo, p1hi, u1lo, u1hi, b1.reshape(1, F),
                     W2[:, :FH], W2[:, FH:])

    p2lo = _spmm_sc(u2lo, src, dst, w3)
    p2hi = _spmm_sc(u2hi, src, dst, w3)
    sums, cnts = _k6(dis, p2lo, p2hi, u2lo, u2hi, b2.reshape(1, F),
                     Batching.reshape(N, 1).astype(jnp.int32))

    chi, rp = _k7(sums, cnts,
                  cW1, cb1.reshape(1, -1), cW2, cb2.reshape(1, -1),
                  cW3, cb3.reshape(1, -1),
                  rW1, rb1.reshape(1, -1), rW2, rb2.reshape(1, -1),
                  rW3, rb3.reshape(1, -1))
    return jnp.concatenate((chi, rp), axis=1)


# speedup vs baseline: 7.8570x; 1.0921x over previous
import functools

import jax
import jax.numpy as jnp
from jax import lax
from jax.experimental import pallas as pl
from jax.experimental.pallas import tpu as pltpu
from jax.experimental.pallas import tpu_sc as plsc

N = 10000
E = 320000
F = 128
G = 64
NC = 2
NS = 16
NW = NC * NS
EC = E // NW
KB = 80
NBLK = EC // KB
RPT = N // NS
ZR = 125

_MESH = plsc.VectorSubcoreMesh(core_axis_name="c", subcore_axis_name="s")


@functools.partial(
    pl.kernel,
    out_type=jax.ShapeDtypeStruct((NW, N), jnp.float32),
    mesh=_MESH,
    compiler_params=pltpu.CompilerParams(use_tc_tiling_on_sc=False),
    scratch_types=[
        pltpu.VMEM((EC,), jnp.int32),
        pltpu.VMEM((EC,), jnp.float32),
        pltpu.VMEM((N + 16,), jnp.float32),
    ],
)
def _deg_sc(dst_hbm, w_hbm, out_hbm, didx, wbuf, acc):
    c = lax.axis_index("c")
    s = lax.axis_index("s")
    wid = s * NC + c

    def zero(i, _):
        acc[pl.ds(i * 16, 16)] = jnp.zeros((16,), jnp.float32)
        return 0

    lax.fori_loop(0, (N + 16) // 16, zero, 0)
    pltpu.sync_copy(dst_hbm.at[wid], didx)
    pltpu.sync_copy(w_hbm.at[wid], wbuf)

    lane0 = lax.iota(jnp.int32, 16) == 0

    def edge16(e, _):
        dvec = didx[pl.ds(e * 16, 16)]
        wvec = wbuf[pl.ds(e * 16, 16)]
        for lane in range(16):
            d = dvec[lane]
            inc = jnp.where(lane0, wvec[lane], 0.0)
            acc[pl.ds(d, 16)] = acc[pl.ds(d, 16)] + inc
        return 0

    lax.fori_loop(0, EC // 16, edge16, 0)
    pltpu.sync_copy(acc.at[pl.ds(0, N)], out_hbm.at[wid])


FH = F // 2


@functools.partial(
    pl.kernel,
    out_type=jax.ShapeDtypeStruct((NC, N, FH), jnp.float32),
    mesh=_MESH,
    compiler_params=pltpu.CompilerParams(use_tc_tiling_on_sc=False),
    scratch_types=[
        pltpu.VMEM((NBLK, KB), jnp.int32),
        pltpu.VMEM((NBLK, KB), jnp.int32),
        pltpu.VMEM((NBLK, KB), jnp.float32),
        pltpu.VMEM((KB, FH), jnp.float32),
        pltpu.VMEM((KB, FH), jnp.float32),
        pltpu.VMEM((ZR, FH), jnp.float32),
        pltpu.VMEM_SHARED((N, FH), jnp.float32),
        pltpu.SemaphoreType.DMA,
        pltpu.SemaphoreType.DMA,
        pltpu.SemaphoreType.DMA,
        pltpu.SemaphoreType.DMA,
    ],
)
def _spmm_sc(u_hbm, src_hbm, dst_hbm, w_hbm, out_hbm,
             sidx, didx, wblk, rows0, rows1, zrows, accum,
             sg0, sg1, ss0, ss1):
    c = lax.axis_index("c")
    s = lax.axis_index("s")
    wid = s * NC + c

    def zrow(i, _):
        for j in range(FH // 16):
            zrows[i, pl.ds(j * 16, 16)] = jnp.zeros((16,), jnp.float32)
        return 0

    lax.fori_loop(0, ZR, zrow, 0)
    for j in range(RPT // ZR):
        pltpu.sync_copy(zrows, accum.at[pl.ds(s * RPT + j * ZR, ZR)])
    plsc.subcore_barrier()

    pltpu.sync_copy(src_hbm.at[wid], sidx)
    pltpu.sync_copy(dst_hbm.at[wid], didx)
    pltpu.sync_copy(w_hbm.at[wid], wblk)

    def fire_gather(b, buf, sem):
        pltpu.async_copy(u_hbm.at[sidx.at[b]], buf, sem)

    def wait_gather(buf, sem):
        pltpu.make_async_copy(u_hbm.at[sidx.at[0]], buf, sem).wait()

    def fire_scatter(b, buf, sem):
        pltpu.async_copy(buf, accum.at[didx.at[b]], sem, add=True)

    def wait_scatter(buf, sem):
        pltpu.make_async_copy(buf, accum.at[didx.at[0]], sem).wait()

    def scale(b, buf):
        def scale16(r16, _):
            wvec = wblk[b, pl.ds(r16 * 16, 16)]
            for lane in range(16):
                wv = wvec[lane]
                r = r16 * 16 + lane
                for j in range(FH // 16):
                    buf[r, pl.ds(j * 16, 16)] = buf[r, pl.ds(j * 16, 16)] * wv
            return 0

        lax.fori_loop(0, KB // 16, scale16, 0)

    fire_gather(0, rows0, sg0)

    def two_blocks(t, _):
        b0 = 2 * t
        wait_gather(rows0, sg0)
        scale(b0, rows0)

        @pl.when(t > 0)
        def _():
            wait_scatter(rows1, ss1)

        fire_gather(b0 + 1, rows1, sg1)
        fire_scatter(b0, rows0, ss0)
        wait_gather(rows1, sg1)
        scale(b0 + 1, rows1)
        wait_scatter(rows0, ss0)
        fire_gather(b0 + 2, rows0, sg0)
        fire_scatter(b0 + 1, rows1, ss1)
        return 0

    lax.fori_loop(0, (NBLK - 1) // 2, two_blocks, 0)
    wait_gather(rows0, sg0)
    scale(NBLK - 1, rows0)
    fire_scatter(NBLK - 1, rows0, ss0)
    wait_scatter(rows1, ss1)
    wait_scatter(rows0, ss0)

    plsc.subcore_barrier()
    pltpu.sync_copy(accum.at[pl.ds(s * RPT, RPT)],
                    out_hbm.at[c, pl.ds(s * RPT, RPT)])


_BN = 1000


def _k2_body(degsT_ref, x_ref, wlo_ref, whi_ref, ulo_ref, uhi_ref, dis_ref):
    deg = jnp.sum(degsT_ref[...], axis=1, keepdims=True) + 1.0
    dis = lax.rsqrt(deg)
    dis_ref[...] = dis
    x = x_ref[...]
    ulo_ref[...] = dis * jnp.dot(x, wlo_ref[...],
                                 preferred_element_type=jnp.float32)
    uhi_ref[...] = dis * jnp.dot(x, whi_ref[...],
                                 preferred_element_type=jnp.float32)


_k2 = pl.pallas_call(
    _k2_body,
    grid=(N // _BN,),
    in_specs=[
        pl.BlockSpec((_BN, NW), lambda i: (i, 0)),
        pl.BlockSpec((_BN, F), lambda i: (i, 0)),
        pl.BlockSpec((F, FH), lambda i: (0, 0)),
        pl.BlockSpec((F, FH), lambda i: (0, 0)),
    ],
    out_specs=[
        pl.BlockSpec((_BN, FH), lambda i: (i, 0)),
        pl.BlockSpec((_BN, FH), lambda i: (i, 0)),
        pl.BlockSpec((_BN, 1), lambda i: (i, 0)),
    ],
    out_shape=[
        jax.ShapeDtypeStruct((N, FH), jnp.float32),
        jax.ShapeDtypeStruct((N, FH), jnp.float32),
        jax.ShapeDtypeStruct((N, 1), jnp.float32),
    ],
)


def _leaky(x):
    return jnp.where(x >= 0, x, 0.01 * x)


def _h_from_halves(dis, plo_ref, phi_ref, ulo_ref, uhi_ref, b_ref):
    hlo = dis * (plo_ref[0] + plo_ref[1] + ulo_ref[...])
    hhi = dis * (phi_ref[0] + phi_ref[1] + uhi_ref[...])
    pre = jnp.concatenate((hlo, hhi), axis=1) + b_ref[...]
    return _leaky(pre)


def _k4_body(dis_ref, plo_ref, phi_ref, ulo_ref, uhi_ref, b_ref,
             w2lo_ref, w2hi_ref, u2lo_ref, u2hi_ref):
    dis = dis_ref[...]
    h = _h_from_halves(dis, plo_ref, phi_ref, ulo_ref, uhi_ref, b_ref)
    u2lo_ref[...] = dis * jnp.dot(h, w2lo_ref[...],
                                  preferred_element_type=jnp.float32)
    u2hi_ref[...] = dis * jnp.dot(h, w2hi_ref[...],
                                  preferred_element_type=jnp.float32)


_k4 = pl.pallas_call(
    _k4_body,
    grid=(N // _BN,),
    in_specs=[
        pl.BlockSpec((_BN, 1), lambda i: (i, 0)),
        pl.BlockSpec((NC, _BN, FH), lambda i: (0, i, 0)),
        pl.BlockSpec((NC, _BN, FH), lambda i: (0, i, 0)),
        pl.BlockSpec((_BN, FH), lambda i: (i, 0)),
        pl.BlockSpec((_BN, FH), lambda i: (i, 0)),
        pl.BlockSpec((1, F), lambda i: (0, 0)),
        pl.BlockSpec((F, FH), lambda i: (0, 0)),
        pl.BlockSpec((F, FH), lambda i: (0, 0)),
    ],
    out_specs=[
        pl.BlockSpec((_BN, FH), lambda i: (i, 0)),
        pl.BlockSpec((_BN, FH), lambda i: (i, 0)),
    ],
    out_shape=[
        jax.ShapeDtypeStruct((N, FH), jnp.float32),
        jax.ShapeDtypeStruct((N, FH), jnp.float32),
    ],
)


def _k6_body(dis_ref, plo_ref, phi_ref, ulo_ref, uhi_ref, b_ref, bat_ref,
             sums_ref, cnts_ref):
    i = pl.program_id(0)
    dis = dis_ref[...]
    h = _h_from_halves(dis, plo_ref, phi_ref, ulo_ref, uhi_ref, b_ref)
    gids = lax.broadcasted_iota(jnp.int32, (_BN, G), 1)
    oh = (bat_ref[...] == gids).astype(jnp.float32)
    psum = lax.dot_general(oh, h, (((0,), (0,)), ((), ())),
                           preferred_element_type=jnp.float32)
    pcnt = lax.dot_general(oh, jnp.ones((_BN, 1), jnp.float32),
                           (((0,), (0,)), ((), ())),
                           preferred_element_type=jnp.float32)

    @pl.when(i == 0)
    def _():
        sums_ref[...] = jnp.zeros_like(sums_ref)
        cnts_ref[...] = jnp.zeros_like(cnts_ref)

    sums_ref[...] += psum
    cnts_ref[...] += pcnt


_k6 = pl.pallas_call(
    _k6_body,
    grid=(N // _BN,),
    in_specs=[
        pl.BlockSpec((_BN, 1), lambda i: (i, 0)),
        pl.BlockSpec((NC, _BN, FH), lambda i: (0, i, 0)),
        pl.BlockSpec((NC, _BN, FH), lambda i: (0, i, 0)),
        pl.BlockSpec((_BN, FH), lambda i: (i, 0)),
        pl.BlockSpec((_BN, FH), lambda i: (i, 0)),
        pl.BlockSpec((1, F), lambda i: (0, 0)),
        pl.BlockSpec((_BN, 1), lambda i: (i, 0)),
    ],
    out_specs=[
        pl.BlockSpec((G, F), lambda i: (0, 0)),
        pl.BlockSpec((G, 1), lambda i: (0, 0)),
    ],
    out_shape=[
        jax.ShapeDtypeStruct((G, F), jnp.float32),
        jax.ShapeDtypeStruct((G, 1), jnp.float32),
    ],
)


def _k7_body(sums_ref, cnts_ref,
             cw1, cb1, cw2, cb2, cw3, cb3,
             rw1, rb1, rw2, rb2, rw3, rb3,
             chi_ref, rp_ref):
    pooled = sums_ref[...] / jnp.maximum(cnts_ref[...], 1.0)

    def head(W1r, B1r, W2r, B2r, W3r, B3r):
        a = jnp.dot(pooled, W1r[...], preferred_element_type=jnp.float32)
        a = _leaky(a + B1r[...])
        a = jnp.dot(a, W2r[...], preferred_element_type=jnp.float32)
        a = _leaky(a + B2r[...])
        return jnp.dot(a, W3r[...], preferred_element_type=jnp.float32) + B3r[...]

    chi_ref[...] = head(cw1, cb1, cw2, cb2, cw3, cb3)
    rp_ref[...] = head(rw1, rb1, rw2, rb2, rw3, rb3)


_k7 = pl.pallas_call(
    _k7_body,
    out_shape=[
        jax.ShapeDtypeStruct((G, 1), jnp.float32),
        jax.ShapeDtypeStruct((G, 1), jnp.float32),
    ],
)


def kernel(X, Edge_index, Edge_weight, Batching,
           W1, b1, W2, b2,
           cW1, cb1, cW2, cb2, cW3, cb3,
           rW1, rb1, rW2, rb2, rW3, rb3):
    src = Edge_index[0].reshape(NW, NBLK, KB)
    dst = Edge_index[1].reshape(NW, NBLK, KB)
    w3 = Edge_weight.reshape(NW, NBLK, KB)
    dstf = Edge_index[1].reshape(NW, EC)
    wf = Edge_weight.reshape(NW, EC)

    degs = _deg_sc(dstf, wf)
    degsT = degs.T
    u1lo, u1hi, dis = _k2(degsT, X, W1[:, :FH], W1[:, FH:])

    p1lo = _spmm_sc(u1lo, src, dst, w3)
    p1hi = _spmm_sc(u1hi, src, dst, w3)
    u2lo, u2hi = _k4(dis, p1lo, p1hi, u1lo, u1hi, b1.reshape(1, F),
                     W2[:, :FH], W2[:, FH:])

    p2lo = _spmm_sc(u2lo, src, dst, w3)
    p2hi = _spmm_sc(u2hi, src, dst, w3)
    sums, cnts = _k6(dis, p2lo, p2hi, u2lo, u2hi, b2.reshape(1, F),
                     Batching.reshape(N, 1).astype(jnp.int32))

    chi, rp = _k7(sums, cnts,
                  cW1, cb1.reshape(1, -1), cW2, cb2.reshape(1, -1),
                  cW3, cb3.reshape(1, -1),
                  rW1, rb1.reshape(1, -1), rW2, rb2.reshape(1, -1),
                  rW3, rb3.reshape(1, -1))
    return jnp.concatenate((chi, rp), axis=1)


# speedup vs baseline: 10.0012x; 1.2729x over previous
import functools

import jax
import jax.numpy as jnp
from jax import lax
from jax.experimental import pallas as pl
from jax.experimental.pallas import tpu as pltpu
from jax.experimental.pallas import tpu_sc as plsc

N = 10000
E = 320000
F = 128
G = 64
NC = 2
NS = 16
NW = NC * NS
EC = E // NW
KB = 80
NBLK = EC // KB
RPT = N // NS
ZR = 125

_MESH = plsc.VectorSubcoreMesh(core_axis_name="c", subcore_axis_name="s")


@functools.partial(
    pl.kernel,
    out_type=jax.ShapeDtypeStruct((NW, N), jnp.float32),
    mesh=_MESH,
    compiler_params=pltpu.CompilerParams(use_tc_tiling_on_sc=False),
    scratch_types=[
        pltpu.VMEM((EC,), jnp.int32),
        pltpu.VMEM((EC,), jnp.float32),
        pltpu.VMEM((N + 16,), jnp.float32),
    ],
)
def _deg_sc(dst_hbm, w_hbm, out_hbm, didx, wbuf, acc):
    c = lax.axis_index("c")
    s = lax.axis_index("s")
    wid = s * NC + c

    def zero(i, _):
        acc[pl.ds(i * 16, 16)] = jnp.zeros((16,), jnp.float32)
        return 0

    lax.fori_loop(0, (N + 16) // 16, zero, 0)
    pltpu.sync_copy(dst_hbm.at[wid], didx)
    pltpu.sync_copy(w_hbm.at[wid], wbuf)

    lane0 = lax.iota(jnp.int32, 16) == 0

    def edge16(e, _):
        dvec = didx[pl.ds(e * 16, 16)]
        wvec = wbuf[pl.ds(e * 16, 16)]
        for lane in range(16):
            d = dvec[lane]
            inc = jnp.where(lane0, wvec[lane], 0.0)
            acc[pl.ds(d, 16)] = acc[pl.ds(d, 16)] + inc
        return 0

    lax.fori_loop(0, EC // 16, edge16, 0)
    pltpu.sync_copy(acc.at[pl.ds(0, N)], out_hbm.at[wid])


FH = F // 2


@functools.partial(
    pl.kernel,
    out_type=jax.ShapeDtypeStruct((NC, N, FH), jnp.float32),
    mesh=_MESH,
    compiler_params=pltpu.CompilerParams(use_tc_tiling_on_sc=False),
    scratch_types=[
        pltpu.VMEM((NBLK, KB), jnp.int32),
        pltpu.VMEM((NBLK, KB), jnp.int32),
        pltpu.VMEM((NBLK, KB), jnp.float32),
        pltpu.VMEM((KB, FH), jnp.float32),
        pltpu.VMEM((KB, FH), jnp.float32),
        pltpu.VMEM((KB, FH), jnp.float32),
        pltpu.VMEM((ZR, FH), jnp.float32),
        pltpu.VMEM_SHARED((N, FH), jnp.float32),
        [pltpu.SemaphoreType.DMA] * 3,
        [pltpu.SemaphoreType.DMA] * 3,
    ],
)
def _spmm_sc(u_hbm, src_hbm, dst_hbm, w_hbm, out_hbm,
             sidx, didx, wblk, rows0, rows1, rows2, zrows, accum, sg, ss):
    c = lax.axis_index("c")
    s = lax.axis_index("s")
    wid = s * NC + c
    bufs = (rows0, rows1, rows2)

    def zrow(i, _):
        for j in range(FH // 16):
            zrows[i, pl.ds(j * 16, 16)] = jnp.zeros((16,), jnp.float32)
        return 0

    lax.fori_loop(0, ZR, zrow, 0)
    for j in range(RPT // ZR):
        pltpu.sync_copy(zrows, accum.at[pl.ds(s * RPT + j * ZR, ZR)])
    plsc.subcore_barrier()

    pltpu.sync_copy(src_hbm.at[wid], sidx)
    pltpu.sync_copy(dst_hbm.at[wid], didx)
    pltpu.sync_copy(w_hbm.at[wid], wblk)

    def fire_gather(b, k):
        pltpu.async_copy(u_hbm.at[sidx.at[b]], bufs[k], sg[k])

    def wait_gather(k):
        pltpu.make_async_copy(u_hbm.at[sidx.at[0]], bufs[k], sg[k]).wait()

    def fire_scatter(b, k):
        pltpu.async_copy(bufs[k], accum.at[didx.at[b]], ss[k], add=True)

    def wait_scatter(k):
        pltpu.make_async_copy(bufs[k], accum.at[didx.at[0]], ss[k]).wait()

    def scale(b, buf):
        def scale16(r16, _):
            wvec = wblk[b, pl.ds(r16 * 16, 16)]
            for lane in range(16):
                wv = wvec[lane]
                r = r16 * 16 + lane
                for j in range(FH // 16):
                    buf[r, pl.ds(j * 16, 16)] = buf[r, pl.ds(j * 16, 16)] * wv
            return 0

        lax.fori_loop(0, KB // 16, scale16, 0)

    fire_gather(0, 0)
    fire_gather(1, 1)

    def three_blocks(t, _):
        for k in range(3):
            bk = 3 * t + k
            kf = (k + 2) % 3

            @pl.when(jnp.logical_and(bk - 1 >= 0, bk - 1 < NBLK))
            def _():
                wait_scatter(kf)

            @pl.when(bk + 2 < NBLK)
            def _():
                fire_gather(bk + 2, kf)

            @pl.when(bk < NBLK)
            def _():
                wait_gather(k)
                scale(bk, bufs[k])
                fire_scatter(bk, k)

        return 0

    lax.fori_loop(0, (NBLK + 2) // 3, three_blocks, 0)

    plsc.subcore_barrier()
    pltpu.sync_copy(accum.at[pl.ds(s * RPT, RPT)],
                    out_hbm.at[c, pl.ds(s * RPT, RPT)])


_BN = 1000


def _k2_body(degsT_ref, x_ref, wlo_ref, whi_ref, ulo_ref, uhi_ref, dis_ref):
    deg = jnp.sum(degsT_ref[...], axis=1, keepdims=True) + 1.0
    dis = lax.rsqrt(deg)
    dis_ref[...] = dis
    x = x_ref[...]
    ulo_ref[...] = dis * jnp.dot(x, wlo_ref[...],
                                 preferred_element_type=jnp.float32)
    uhi_ref[...] = dis * jnp.dot(x, whi_ref[...],
                                 preferred_element_type=jnp.float32)


_k2 = pl.pallas_call(
    _k2_body,
    grid=(N // _BN,),
    in_specs=[
        pl.BlockSpec((_BN, NW), lambda i: (i, 0)),
        pl.BlockSpec((_BN, F), lambda i: (i, 0)),
        pl.BlockSpec((F, FH), lambda i: (0, 0)),
        pl.BlockSpec((F, FH), lambda i: (0, 0)),
    ],
    out_specs=[
        pl.BlockSpec((_BN, FH), lambda i: (i, 0)),
        pl.BlockSpec((_BN, FH), lambda i: (i, 0)),
        pl.BlockSpec((_BN, 1), lambda i: (i, 0)),
    ],
    out_shape=[
        jax.ShapeDtypeStruct((N, FH), jnp.float32),
        jax.ShapeDtypeStruct((N, FH), jnp.float32),
        jax.ShapeDtypeStruct((N, 1), jnp.float32),
    ],
)


def _leaky(x):
    return jnp.where(x >= 0, x, 0.01 * x)


def _h_from_halves(dis, plo_ref, phi_ref, ulo_ref, uhi_ref, b_ref):
    hlo = dis * (plo_ref[0] + plo_ref[1] + ulo_ref[...])
    hhi = dis * (phi_ref[0] + phi_ref[1] + uhi_ref[...])
    pre = jnp.concatenate((hlo, hhi), axis=1) + b_ref[...]
    return _leaky(pre)


def _k4_body(dis_ref, plo_ref, phi_ref, ulo_ref, uhi_ref, b_ref,
             w2lo_ref, w2hi_ref, u2lo_ref, u2hi_ref):
    dis = dis_ref[...]
    h = _h_from_halves(dis, plo_ref, phi_ref, ulo_ref, uhi_ref, b_ref)
    u2lo_ref[...] = dis * jnp.dot(h, w2lo_ref[...],
                                  preferred_element_type=jnp.float32)
    u2hi_ref[...] = dis * jnp.dot(h, w2hi_ref[...],
                                  preferred_element_type=jnp.float32)


_k4 = pl.pallas_call(
    _k4_body,
    grid=(N // _BN,),
    in_specs=[
        pl.BlockSpec((_BN, 1), lambda i: (i, 0)),
        pl.BlockSpec((NC, _BN, FH), lambda i: (0, i, 0)),
        pl.BlockSpec((NC, _BN, FH), lambda i: (0, i, 0)),
        pl.BlockSpec((_BN, FH), lambda i: (i, 0)),
        pl.BlockSpec((_BN, FH), lambda i: (i, 0)),
        pl.BlockSpec((1, F), lambda i: (0, 0)),
        pl.BlockSpec((F, FH), lambda i: (0, 0)),
        pl.BlockSpec((F, FH), lambda i: (0, 0)),
    ],
    out_specs=[
        pl.BlockSpec((_BN, FH), lambda i: (i, 0)),
        pl.BlockSpec((_BN, FH), lambda i: (i, 0)),
    ],
    out_shape=[
        jax.ShapeDtypeStruct((N, FH), jnp.float32),
        jax.ShapeDtypeStruct((N, FH), jnp.float32),
    ],
)


def _k6_body(dis_ref, plo_ref, phi_ref, ulo_ref, uhi_ref, b_ref, bat_ref,
             sums_ref, cnts_ref):
    i = pl.program_id(0)
    dis = dis_ref[...]
    h = _h_from_halves(dis, plo_ref, phi_ref, ulo_ref, uhi_ref, b_ref)
    gids = lax.broadcasted_iota(jnp.int32, (_BN, G), 1)
    oh = (bat_ref[...] == gids).astype(jnp.float32)
    psum = lax.dot_general(oh, h, (((0,), (0,)), ((), ())),
                           preferred_element_type=jnp.float32)
    pcnt = lax.dot_general(oh, jnp.ones((_BN, 1), jnp.float32),
                           (((0,), (0,)), ((), ())),
                           preferred_element_type=jnp.float32)

    @pl.when(i == 0)
    def _():
        sums_ref[...] = jnp.zeros_like(sums_ref)
        cnts_ref[...] = jnp.zeros_like(cnts_ref)

    sums_ref[...] += psum
    cnts_ref[...] += pcnt


_k6 = pl.pallas_call(
    _k6_body,
    grid=(N // _BN,),
    in_specs=[
        pl.BlockSpec((_BN, 1), lambda i: (i, 0)),
        pl.BlockSpec((NC, _BN, FH), lambda i: (0, i, 0)),
        pl.BlockSpec((NC, _BN, FH), lambda i: (0, i, 0)),
        pl.BlockSpec((_BN, FH), lambda i: (i, 0)),
        pl.BlockSpec((_BN, FH), lambda i: (i, 0)),
        pl.BlockSpec((1, F), lambda i: (0, 0)),
        pl.BlockSpec((_BN, 1), lambda i: (i, 0)),
    ],
    out_specs=[
        pl.BlockSpec((G, F), lambda i: (0, 0)),
        pl.BlockSpec((G, 1), lambda i: (0, 0)),
    ],
    out_shape=[
        jax.ShapeDtypeStruct((G, F), jnp.float32),
        jax.ShapeDtypeStruct((G, 1), jnp.float32),
    ],
)


def _k7_body(sums_ref, cnts_ref,
             cw1, cb1, cw2, cb2, cw3, cb3,
             rw1, rb1, rw2, rb2, rw3, rb3,
             chi_ref, rp_ref):
    pooled = sums_ref[...] / jnp.maximum(cnts_ref[...], 1.0)

    def head(W1r, B1r, W2r, B2r, W3r, B3r):
        a = jnp.dot(pooled, W1r[...], preferred_element_type=jnp.float32)
        a = _leaky(a + B1r[...])
        a = jnp.dot(a, W2r[...], preferred_element_type=jnp.float32)
        a = _leaky(a + B2r[...])
        return jnp.dot(a, W3r[...], preferred_element_type=jnp.float32) + B3r[...]

    chi_ref[...] = head(cw1, cb1, cw2, cb2, cw3, cb3)
    rp_ref[...] = head(rw1, rb1, rw2, rb2, rw3, rb3)


_k7 = pl.pallas_call(
    _k7_body,
    out_shape=[
        jax.ShapeDtypeStruct((G, 1), jnp.float32),
        jax.ShapeDtypeStruct((G, 1), jnp.float32),
    ],
)


def kernel(X, Edge_index, Edge_weight, Batching,
           W1, b1, W2, b2,
           cW1, cb1, cW2, cb2, cW3, cb3,
           rW1, rb1, rW2, rb2, rW3, rb3):
    src = Edge_index[0].reshape(NW, NBLK, KB)
    dst = Edge_index[1].reshape(NW, NBLK, KB)
    w3 = Edge_weight.reshape(NW, NBLK, KB)
    dstf = Edge_index[1].reshape(NW, EC)
    wf = Edge_weight.reshape(NW, EC)

    degs = _deg_sc(dstf, wf)
    degsT = degs.T
    u1lo, u1hi, dis = _k2(degsT, X, W1[:, :FH], W1[:, FH:])

    p1lo = _spmm_sc(u1lo, src, dst, w3)
    p1hi = _spmm_sc(u1hi, src, dst, w3)
    u2lo, u2hi = _k4(dis, p1lo, p1hi, u1lo, u1hi, b1.reshape(1, F),
                     W2[:, :FH], W2[:, FH:])

    p2lo = _spmm_sc(u2lo, src, dst, w3)
    p2hi = _spmm_sc(u2hi, src, dst, w3)
    sums, cnts = _k6(dis, p2lo, p2hi, u2lo, u2hi, b2.reshape(1, F),
                     Batching.reshape(N, 1).astype(jnp.int32))

    chi, rp = _k7(sums, cnts,
                  cW1, cb1.reshape(1, -1), cW2, cb2.reshape(1, -1),
                  cW3, cb3.reshape(1, -1),
                  rW1, rb1.reshape(1, -1), rW2, rb2.reshape(1, -1),
                  rW3, rb3.reshape(1, -1))
    return jnp.concatenate((chi, rp), axis=1)


# speedup vs baseline: 21.0888x; 2.1086x over previous
import functools

import jax
import jax.numpy as jnp
from jax import lax
from jax.experimental import pallas as pl
from jax.experimental.pallas import tpu as pltpu
from jax.experimental.pallas import tpu_sc as plsc

N = 10000
E = 320000
F = 128
G = 64
NC = 2
NS = 16
NW = NC * NS
EC = E // NW
KB = 80
NBLK = EC // KB
RPT = N // NS
ZR = 125

_MESH = plsc.VectorSubcoreMesh(core_axis_name="c", subcore_axis_name="s")


@functools.partial(
    pl.kernel,
    out_type=jax.ShapeDtypeStruct((NW, N), jnp.float32),
    mesh=_MESH,
    compiler_params=pltpu.CompilerParams(use_tc_tiling_on_sc=False),
    scratch_types=[
        pltpu.VMEM((EC,), jnp.int32),
        pltpu.VMEM((EC,), jnp.float32),
        pltpu.VMEM((N + 16,), jnp.float32),
    ],
)
def _deg_sc(dst_hbm, w_hbm, out_hbm, didx, wbuf, acc):
    c = lax.axis_index("c")
    s = lax.axis_index("s")
    wid = s * NC + c

    def zero(i, _):
        acc[pl.ds(i * 16, 16)] = jnp.zeros((16,), jnp.float32)
        return 0

    lax.fori_loop(0, (N + 16) // 16, zero, 0)
    pltpu.sync_copy(dst_hbm.at[wid], didx)
    pltpu.sync_copy(w_hbm.at[wid], wbuf)

    lane0 = lax.iota(jnp.int32, 16) == 0

    def edge16(e, _):
        dvec = didx[pl.ds(e * 16, 16)]
        wvec = wbuf[pl.ds(e * 16, 16)]
        for lane in range(16):
            d = dvec[lane]
            inc = jnp.where(lane0, wvec[lane], 0.0)
            acc[pl.ds(d, 16)] = acc[pl.ds(d, 16)] + inc
        return 0

    lax.fori_loop(0, EC // 16, edge16, 0)
    pltpu.sync_copy(acc.at[pl.ds(0, N)], out_hbm.at[wid])


FH = F // 2


@functools.partial(
    pl.kernel,
    out_type=jax.ShapeDtypeStruct((NC, N, FH), jnp.float32),
    mesh=_MESH,
    compiler_params=pltpu.CompilerParams(use_tc_tiling_on_sc=False),
    scratch_types=[
        pltpu.VMEM((NBLK, KB), jnp.int32),
        pltpu.VMEM((NBLK, KB), jnp.int32),
        pltpu.VMEM((NBLK, KB), jnp.float32),
        [pltpu.VMEM((KB, FH), jnp.float32)] * 3,
        [pltpu.VMEM((KB, FH), jnp.float32)] * 3,
        pltpu.VMEM((ZR, FH), jnp.float32),
        pltpu.VMEM_SHARED((N, FH), jnp.float32),
        [pltpu.SemaphoreType.DMA] * 3,
        [pltpu.SemaphoreType.DMA] * 3,
    ],
)
def _spmm_sc(u_hbm, src_hbm, dst_hbm, w_hbm, out_hbm,
             sidx, didx, wblk, bufs, sbufs, zrows, accum, sg, ss):
    c = lax.axis_index("c")
    s = lax.axis_index("s")
    wid = s * NC + c

    def zrow(i, _):
        for j in range(FH // 16):
            zrows[i, pl.ds(j * 16, 16)] = jnp.zeros((16,), jnp.float32)
        return 0

    lax.fori_loop(0, ZR, zrow, 0)
    for j in range(RPT // ZR):
        pltpu.sync_copy(zrows, accum.at[pl.ds(s * RPT + j * ZR, ZR)])
    plsc.subcore_barrier()

    pltpu.sync_copy(src_hbm.at[wid], sidx)
    pltpu.sync_copy(dst_hbm.at[wid], didx)
    pltpu.sync_copy(w_hbm.at[wid], wblk)

    def fire_gather(b, k):
        pltpu.async_copy(u_hbm.at[sidx.at[b]], bufs[k], sg[k])

    def wait_gather(k):
        pltpu.make_async_copy(u_hbm.at[sidx.at[0]], bufs[k], sg[k]).wait()

    def fire_scatter(b, k):
        pltpu.async_copy(sbufs[k], accum.at[didx.at[b]], ss[k], add=True)

    def wait_scatter(k):
        pltpu.make_async_copy(sbufs[k], accum.at[didx.at[0]], ss[k]).wait()

    def scale(b, src_buf, dst_buf):
        def scale16(r16, _):
            wvec = wblk[b, pl.ds(r16 * 16, 16)]
            for lane in range(16):
                wv = wvec[lane]
                r = r16 * 16 + lane
                for j in range(FH // 16):
                    dst_buf[r, pl.ds(j * 16, 16)] = (
                        src_buf[r, pl.ds(j * 16, 16)] * wv)
            return 0

        lax.fori_loop(0, KB // 16, scale16, 0)

    fire_gather(0, 0)
    fire_gather(1, 1)

    def three_blocks(t, _):
        for k in range(3):
            bk = 3 * t + k
            kf = (k + 2) % 3

            @pl.when(jnp.logical_and(bk - 1 >= 0, bk - 1 < NBLK))
            def _():
                wait_scatter(kf)

            @pl.when(bk + 2 < NBLK)
            def _():
                fire_gather(bk + 2, kf)

            @pl.when(bk < NBLK)
            def _():
                wait_gather(k)
                scale(bk, bufs[k], sbufs[k])
                fire_scatter(bk, k)

        return 0

    lax.fori_loop(0, (NBLK + 2) // 3, three_blocks, 0)

    plsc.subcore_barrier()
    pltpu.sync_copy(accum.at[pl.ds(s * RPT, RPT)],
                    out_hbm.at[c, pl.ds(s * RPT, RPT)])


_BN = 1000


def _k2_body(degsT_ref, x_ref, wlo_ref, whi_ref, ulo_ref, uhi_ref, dis_ref):
    deg = jnp.sum(degsT_ref[...], axis=1, keepdims=True) + 1.0
    dis = lax.rsqrt(deg)
    dis_ref[...] = dis
    x = x_ref[...]
    ulo_ref[...] = dis * jnp.dot(x, wlo_ref[...],
                                 preferred_element_type=jnp.float32)
    uhi_ref[...] = dis * jnp.dot(x, whi_ref[...],
                                 preferred_element_type=jnp.float32)


_k2 = pl.pallas_call(
    _k2_body,
    grid=(N // _BN,),
    in_specs=[
        pl.BlockSpec((_BN, NW), lambda i: (i, 0)),
        pl.BlockSpec((_BN, F), lambda i: (i, 0)),
        pl.BlockSpec((F, FH), lambda i: (0, 0)),
        pl.BlockSpec((F, FH), lambda i: (0, 0)),
    ],
    out_specs=[
        pl.BlockSpec((_BN, FH), lambda i: (i, 0)),
        pl.BlockSpec((_BN, FH), lambda i: (i, 0)),
        pl.BlockSpec((_BN, 1), lambda i: (i, 0)),
    ],
    out_shape=[
        jax.ShapeDtypeStruct((N, FH), jnp.float32),
        jax.ShapeDtypeStruct((N, FH), jnp.float32),
        jax.ShapeDtypeStruct((N, 1), jnp.float32),
    ],
)


def _leaky(x):
    return jnp.where(x >= 0, x, 0.01 * x)


def _h_from_halves(dis, plo_ref, phi_ref, ulo_ref, uhi_ref, b_ref):
    hlo = dis * (plo_ref[0] + plo_ref[1] + ulo_ref[...])
    hhi = dis * (phi_ref[0] + phi_ref[1] + uhi_ref[...])
    pre = jnp.concatenate((hlo, hhi), axis=1) + b_ref[...]
    return _leaky(pre)


def _k4_body(dis_ref, plo_ref, phi_ref, ulo_ref, uhi_ref, b_ref,
             w2lo_ref, w2hi_ref, u2lo_ref, u2hi_ref):
    dis = dis_ref[...]
    h = _h_from_halves(dis, plo_ref, phi_ref, ulo_ref, uhi_ref, b_ref)
    u2lo_ref[...] = dis * jnp.dot(h, w2lo_ref[...],
                                  preferred_element_type=jnp.float32)
    u2hi_ref[...] = dis * jnp.dot(h, w2hi_ref[...],
                                  preferred_element_type=jnp.float32)


_k4 = pl.pallas_call(
    _k4_body,
    grid=(N // _BN,),
    in_specs=[
        pl.BlockSpec((_BN, 1), lambda i: (i, 0)),
        pl.BlockSpec((NC, _BN, FH), lambda i: (0, i, 0)),
        pl.BlockSpec((NC, _BN, FH), lambda i: (0, i, 0)),
        pl.BlockSpec((_BN, FH), lambda i: (i, 0)),
        pl.BlockSpec((_BN, FH), lambda i: (i, 0)),
        pl.BlockSpec((1, F), lambda i: (0, 0)),
        pl.BlockSpec((F, FH), lambda i: (0, 0)),
        pl.BlockSpec((F, FH), lambda i: (0, 0)),
    ],
    out_specs=[
        pl.BlockSpec((_BN, FH), lambda i: (i, 0)),
        pl.BlockSpec((_BN, FH), lambda i: (i, 0)),
    ],
    out_shape=[
        jax.ShapeDtypeStruct((N, FH), jnp.float32),
        jax.ShapeDtypeStruct((N, FH), jnp.float32),
    ],
)


def _k6_body(dis_ref, plo_ref, phi_ref, ulo_ref, uhi_ref, b_ref, bat_ref,
             sums_ref, cnts_ref):
    i = pl.program_id(0)
    dis = dis_ref[...]
    h = _h_from_halves(dis, plo_ref, phi_ref, ulo_ref, uhi_ref, b_ref)
    gids = lax.broadcasted_iota(jnp.int32, (_BN, G), 1)
    oh = (bat_ref[...] == gids).astype(jnp.float32)
    psum = lax.dot_general(oh, h, (((0,), (0,)), ((), ())),
                           preferred_element_type=jnp.float32)
    pcnt = lax.dot_general(oh, jnp.ones((_BN, 1), jnp.float32),
                           (((0,), (0,)), ((), ())),
                           preferred_element_type=jnp.float32)

    @pl.when(i == 0)
    def _():
        sums_ref[...] = jnp.zeros_like(sums_ref)
        cnts_ref[...] = jnp.zeros_like(cnts_ref)

    sums_ref[...] += psum
    cnts_ref[...] += pcnt


_k6 = pl.pallas_call(
    _k6_body,
    grid=(N // _BN,),
    in_specs=[
        pl.BlockSpec((_BN, 1), lambda i: (i, 0)),
        pl.BlockSpec((NC, _BN, FH), lambda i: (0, i, 0)),
        pl.BlockSpec((NC, _BN, FH), lambda i: (0, i, 0)),
        pl.BlockSpec((_BN, FH), lambda i: (i, 0)),
        pl.BlockSpec((_BN, FH), lambda i: (i, 0)),
        pl.BlockSpec((1, F), lambda i: (0, 0)),
        pl.BlockSpec((_BN, 1), lambda i: (i, 0)),
    ],
    out_specs=[
        pl.BlockSpec((G, F), lambda i: (0, 0)),
        pl.BlockSpec((G, 1), lambda i: (0, 0)),
    ],
    out_shape=[
        jax.ShapeDtypeStruct((G, F), jnp.float32),
        jax.ShapeDtypeStruct((G, 1), jnp.float32),
    ],
)


def _k7_body(sums_ref, cnts_ref,
             cw1, cb1, cw2, cb2, cw3, cb3,
             rw1, rb1, rw2, rb2, rw3, rb3,
             chi_ref, rp_ref):
    pooled = sums_ref[...] / jnp.maximum(cnts_ref[...], 1.0)

    def head(W1r, B1r, W2r, B2r, W3r, B3r):
        a = jnp.dot(pooled, W1r[...], preferred_element_type=jnp.float32)
        a = _leaky(a + B1r[...])
        a = jnp.dot(a, W2r[...], preferred_element_type=jnp.float32)
        a = _leaky(a + B2r[...])
        return jnp.dot(a, W3r[...], preferred_element_type=jnp.float32) + B3r[...]

    chi_ref[...] = head(cw1, cb1, cw2, cb2, cw3, cb3)
    rp_ref[...] = head(rw1, rb1, rw2, rb2, rw3, rb3)


_k7 = pl.pallas_call(
    _k7_body,
    out_shape=[
        jax.ShapeDtypeStruct((G, 1), jnp.float32),
        jax.ShapeDtypeStruct((G, 1), jnp.float32),
    ],
)


def kernel(X, Edge_index, Edge_weight, Batching,
           W1, b1, W2, b2,
           cW1, cb1, cW2, cb2, cW3, cb3,
           rW1, rb1, rW2, rb2, rW3, rb3):
    src = Edge_index[0].reshape(NW, NBLK, KB)
    dst = Edge_index[1].reshape(NW, NBLK, KB)
    w3 = Edge_weight.reshape(NW, NBLK, KB)
    dstf = Edge_index[1].reshape(NW, EC)
    wf = Edge_weight.reshape(NW, EC)

    degs = _deg_sc(dstf, wf)
    degsT = degs.T
    u1lo, u1hi, dis = _k2(degsT, X, W1[:, :FH], W1[:, FH:])

    p1lo = _spmm_sc(u1lo, src, dst, w3)
    p1hi = _spmm_sc(u1hi, src, dst, w3)
    u2lo, u2hi = _k4(dis, p1lo, p1hi, u1lo, u1hi, b1.reshape(1, F),
                     W2[:, :FH], W2[:, FH:])

    p2lo = _spmm_sc(u2lo, src, dst, w3)
    p2hi = _spmm_sc(u2hi, src, dst, w3)
    sums, cnts = _k6(dis, p2lo, p2hi, u2lo, u2hi, b2.reshape(1, F),
                     Batching.reshape(N, 1).astype(jnp.int32))

    chi, rp = _k7(sums, cnts,
                  cW1, cb1.reshape(1, -1), cW2, cb2.reshape(1, -1),
                  cW3, cb3.reshape(1, -1),
                  rW1, rb1.reshape(1, -1), rW2, rb2.reshape(1, -1),
                  rW3, rb3.reshape(1, -1))
    return jnp.concatenate((chi, rp), axis=1)
